# Initial kernel scaffold; baseline (speedup 1.0000x reference)
import functools

import jax
import jax.numpy as jnp
from jax import lax
from jax.experimental import pallas as pl
from jax.experimental.pallas import tpu as pltpu
from jax.experimental.pallas import tpu_sc as plsc

N = 10000
E = 320000
D = 128
P = 512
MLP_DIM = 256
NACT = 10
LAYERS = 3

NC = 2
NS = 16
CH = 128
EPT = E // NS
NCHUNK = (EPT + CH - 1) // CH
EPT_PAD = NCHUNK * CH
NPAD = N + 16
ROWS_PER_TILE = N // NS


def _sc_agg(hs, hf, gidx, sidx, zeros):
  mesh = plsc.VectorSubcoreMesh(core_axis_name="c", subcore_axis_name="s")

  @functools.partial(
      pl.kernel,
      out_type=[jax.ShapeDtypeStruct((N, D), jnp.float32)] * 2,
      mesh=mesh,
      scratch_types=[
          pltpu.VMEM((NCHUNK, CH), jnp.int32),
          pltpu.VMEM((NCHUNK, CH), jnp.int32),
          pltpu.VMEM((CH, D), jnp.float32),
          pltpu.VMEM((CH, D), jnp.float32),
          pltpu.VMEM_SHARED((NPAD, D), jnp.float32),
          pltpu.SemaphoreType.DMA,
          pltpu.SemaphoreType.DMA,
      ],
  )
  def k(hs_hbm, hf_hbm, g_hbm, s_hbm, z_hbm, aggs_hbm, aggf_hbm,
        gv, sv, r0, r1, agg_sh, sem0, sem1):
    c = lax.axis_index("c")
    s = lax.axis_index("s")
    base = s * ROWS_PER_TILE
    pltpu.sync_copy(z_hbm.at[pl.ds(0, ROWS_PER_TILE)],
                    agg_sh.at[pl.ds(base, ROWS_PER_TILE)])
    pltpu.sync_copy(g_hbm.at[c, s], gv)
    pltpu.sync_copy(s_hbm.at[c, s], sv)
    plsc.subcore_barrier()

    def run(h_hbm):
      pltpu.async_copy(h_hbm.at[gv.at[0]], r0, sem0)
      npairs = (NCHUNK - 1) // 2

      def body(i, _):
        j0 = 2 * i
        pltpu.async_copy(h_hbm.at[gv.at[j0 + 1]], r1, sem1)
        pltpu.make_async_copy(h_hbm.at[gv.at[j0]], r0, sem0).wait()
        pltpu.sync_copy(r0, agg_sh.at[sv.at[j0]], add=True)
        pltpu.async_copy(h_hbm.at[gv.at[j0 + 2]], r0, sem0)
        pltpu.make_async_copy(h_hbm.at[gv.at[j0 + 1]], r1, sem1).wait()
        pltpu.sync_copy(r1, agg_sh.at[sv.at[j0 + 1]], add=True)
        return 0

      lax.fori_loop(0, npairs, body, 0)
      pltpu.make_async_copy(h_hbm.at[gv.at[NCHUNK - 1]], r0, sem0).wait()
      pltpu.sync_copy(r0, agg_sh.at[sv.at[NCHUNK - 1]], add=True)

    @pl.when(c == 0)
    def _():
      run(hs_hbm)

    @pl.when(c == 1)
    def _():
      run(hf_hbm)

    plsc.subcore_barrier()

    @pl.when(c == 0)
    def _():
      pltpu.sync_copy(agg_sh.at[pl.ds(base, ROWS_PER_TILE)],
                      aggs_hbm.at[pl.ds(base, ROWS_PER_TILE)])

    @pl.when(c == 1)
    def _():
      pltpu.sync_copy(agg_sh.at[pl.ds(base, ROWS_PER_TILE)],
                      aggf_hbm.at[pl.ds(base, ROWS_PER_TILE)])

  return k(hs, hf, gidx, sidx, zeros)


_BLK = 1250


def _tc_dense_body(aggs_ref, hs_ref, aggf_ref, hf_ref,
                   wns, wss, bs1, wnf, wsf, bf1, os_ref, of_ref):
  os_ref[...] = jnp.maximum(
      jnp.dot(aggs_ref[...], wns[...], preferred_element_type=jnp.float32)
      + jnp.dot(hs_ref[...], wss[...], preferred_element_type=jnp.float32)
      + bs1[...], 0.0)
  of_ref[...] = jnp.maximum(
      jnp.dot(aggf_ref[...], wnf[...], preferred_element_type=jnp.float32)
      + jnp.dot(hf_ref[...], wsf[...], preferred_element_type=jnp.float32)
      + bf1[...], 0.0)


def _tc_dense(aggs, hs, aggf, hf, wns, wss, bs1, wnf, wsf, bf1):
  row_spec = pl.BlockSpec((_BLK, D), lambda i: (i, 0))
  w_spec = pl.BlockSpec((D, D), lambda i: (0, 0))
  b_spec = pl.BlockSpec((1, D), lambda i: (0, 0))
  return pl.pallas_call(
      _tc_dense_body,
      grid=(N // _BLK,),
      in_specs=[row_spec, row_spec, row_spec, row_spec,
                w_spec, w_spec, b_spec, w_spec, w_spec, b_spec],
      out_specs=[row_spec, row_spec],
      out_shape=[jax.ShapeDtypeStruct((N, D), jnp.float32)] * 2,
  )(aggs, hs, aggf, hf, wns, wss, bs1, wnf, wsf, bf1)


_PPT = P // NS


def _sc_po_gather(hs, hf, pos):
  mesh = plsc.VectorSubcoreMesh(core_axis_name="c", subcore_axis_name="s")

  @functools.partial(
      pl.kernel,
      out_type=[jax.ShapeDtypeStruct((P, D), jnp.float32)] * 2,
      mesh=mesh,
      scratch_types=[
          pltpu.VMEM((_PPT,), jnp.int32),
          pltpu.VMEM((_PPT, D), jnp.float32),
          pltpu.SemaphoreType.DMA,
      ],
  )
  def k(hs_hbm, hf_hbm, pos_hbm, embs_hbm, embf_hbm, pidx, rows, sem):
    c = lax.axis_index("c")
    s = lax.axis_index("s")
    base = s * _PPT
    pltpu.sync_copy(pos_hbm.at[pl.ds(base, _PPT)], pidx)

    @pl.when(c == 0)
    def _():
      pltpu.async_copy(hs_hbm.at[pidx], rows, sem).wait()
      pltpu.sync_copy(rows, embs_hbm.at[pl.ds(base, _PPT)])

    @pl.when(c == 1)
    def _():
      pltpu.async_copy(hf_hbm.at[pidx], rows, sem).wait()
      pltpu.sync_copy(rows, embf_hbm.at[pl.ds(base, _PPT)])

  return k(hs, hf, pos)


def _tc_mlp_body(es_ref, ef_ref, w1s, w1f, b1r, w2r, b2r, w3r, b3r, out_ref):
  h = jnp.maximum(
      jnp.dot(es_ref[...], w1s[...], preferred_element_type=jnp.float32)
      + jnp.dot(ef_ref[...], w1f[...], preferred_element_type=jnp.float32)
      + b1r[...], 0.0)
  h = jnp.maximum(
      jnp.dot(h, w2r[...], preferred_element_type=jnp.float32) + b2r[...], 0.0)
  out_ref[...] = (
      jnp.dot(h, w3r[...], preferred_element_type=jnp.float32) + b3r[...])


def _tc_mlp(embs, embf, w1s, w1f, b1, w2, b2, w3p, b3p):
  return pl.pallas_call(
      _tc_mlp_body,
      out_shape=jax.ShapeDtypeStruct((P, 128), jnp.float32),
  )(embs, embf, w1s, w1f, b1, w2, b2, w3p, b3p)


def kernel(x, edge_index, POs, Wn_s, Wself_s, b_s, Wn_f, Wself_f, b_f,
           W1, b1, W2, b2, W3, b3):
  src = edge_index[0]
  dst = edge_index[1]
  pad = NS * EPT_PAD - E
  gpad = jnp.zeros((pad,), jnp.int32)
  spad = jnp.full((pad,), N, jnp.int32)
  gidx = jnp.stack([jnp.concatenate([src, gpad]),
                    jnp.concatenate([dst, gpad])]).reshape(NC, NS, NCHUNK, CH)
  sidx = jnp.stack([jnp.concatenate([dst, spad]),
                    jnp.concatenate([src, spad])]).reshape(NC, NS, NCHUNK, CH)
  zeros = jnp.zeros((ROWS_PER_TILE, D), jnp.float32)

  hs = x
  hf = x
  for l in range(LAYERS):
    aggs, aggf = _sc_agg(hs, hf, gidx, sidx, zeros)
    hs, hf = _tc_dense(aggs, hs, aggf, hf,
                       Wn_s[l], Wself_s[l], b_s[l].reshape(1, D),
                       Wn_f[l], Wself_f[l], b_f[l].reshape(1, D))

  embs, embf = _sc_po_gather(hs, hf, POs)
  w3p = jnp.zeros((MLP_DIM, 128), jnp.float32).at[:, :NACT].set(W3)
  b3p = jnp.zeros((1, 128), jnp.float32).at[:, :NACT].set(b3.reshape(1, NACT))
  y = _tc_mlp(embs, embf, W1[:D], W1[D:], b1.reshape(1, MLP_DIM),
              W2, b2.reshape(1, MLP_DIM), w3p, b3p)
  return y[:, :NACT]


# speedup vs baseline: 2.7297x; 2.7297x over previous
import functools

import jax
import jax.numpy as jnp
from jax import lax
from jax.experimental import pallas as pl
from jax.experimental.pallas import tpu as pltpu
from jax.experimental.pallas import tpu_sc as plsc

N = 10000
E = 320000
D = 128
P = 512
MLP_DIM = 256
NACT = 10
LAYERS = 3

NC = 2
NS = 16
CH = 128
SUP = 20
NSUP = 8
NCHUNK = SUP * NSUP
EPT_PAD = NCHUNK * CH
NPAD = N + 16
RPT = 624
RPT_LAST = N - (NS - 1) * RPT


def _sc_agg(hs, hf, gidx, sidx, zeros):
  mesh = plsc.VectorSubcoreMesh(core_axis_name="c", subcore_axis_name="s")

  @functools.partial(
      pl.kernel,
      out_type=[jax.ShapeDtypeStruct((N, D), jnp.float32)] * 2,
      mesh=mesh,
      scratch_types=[
          pltpu.VMEM((SUP, CH), jnp.int32),
          pltpu.VMEM((SUP, CH), jnp.int32),
          pltpu.VMEM((CH, D), jnp.float32),
          pltpu.VMEM((CH, D), jnp.float32),
          pltpu.VMEM_SHARED((NPAD, D), jnp.float32),
          pltpu.SemaphoreType.DMA,
          pltpu.SemaphoreType.DMA,
      ],
  )
  def k(hs_hbm, hf_hbm, g_hbm, s_hbm, z_hbm, aggs_hbm, aggf_hbm,
        gv, sv, r0, r1, agg_sh, sem0, sem1):
    c = lax.axis_index("c")
    s = lax.axis_index("s")
    base = s * RPT

    @pl.when(s < NS - 1)
    def _():
      pltpu.sync_copy(z_hbm.at[pl.ds(0, RPT)], agg_sh.at[pl.ds(base, RPT)])

    @pl.when(s == NS - 1)
    def _():
      pltpu.sync_copy(z_hbm.at[pl.ds(0, RPT_LAST)],
                      agg_sh.at[pl.ds(base, RPT_LAST)])

    plsc.subcore_barrier()

    def run(h_hbm):
      def superstep(sup, _):
        pltpu.sync_copy(g_hbm.at[c, s, sup], gv)
        pltpu.sync_copy(s_hbm.at[c, s, sup], sv)
        pltpu.async_copy(h_hbm.at[gv.at[0]], r0, sem0)

        def body(i, _):
          j0 = 2 * i
          pltpu.async_copy(h_hbm.at[gv.at[j0 + 1]], r1, sem1)
          pltpu.make_async_copy(h_hbm.at[gv.at[j0]], r0, sem0).wait()
          pltpu.sync_copy(r0, agg_sh.at[sv.at[j0]], add=True)
          pltpu.async_copy(h_hbm.at[gv.at[j0 + 2]], r0, sem0)
          pltpu.make_async_copy(h_hbm.at[gv.at[j0 + 1]], r1, sem1).wait()
          pltpu.sync_copy(r1, agg_sh.at[sv.at[j0 + 1]], add=True)
          return 0

        lax.fori_loop(0, SUP // 2 - 1, body, 0)
        pltpu.async_copy(h_hbm.at[gv.at[SUP - 1]], r1, sem1)
        pltpu.make_async_copy(h_hbm.at[gv.at[SUP - 2]], r0, sem0).wait()
        pltpu.sync_copy(r0, agg_sh.at[sv.at[SUP - 2]], add=True)
        pltpu.make_async_copy(h_hbm.at[gv.at[SUP - 1]], r1, sem1).wait()
        pltpu.sync_copy(r1, agg_sh.at[sv.at[SUP - 1]], add=True)
        return 0

      lax.fori_loop(0, NSUP, superstep, 0)

    @pl.when(c == 0)
    def _():
      run(hs_hbm)

    @pl.when(c == 1)
    def _():
      run(hf_hbm)

    plsc.subcore_barrier()

    def copy_out(out_hbm):
      @pl.when(s < NS - 1)
      def _():
        pltpu.sync_copy(agg_sh.at[pl.ds(base, RPT)],
                        out_hbm.at[pl.ds(base, RPT)])

      @pl.when(s == NS - 1)
      def _():
        pltpu.sync_copy(agg_sh.at[pl.ds(base, RPT_LAST)],
                        out_hbm.at[pl.ds(base, RPT_LAST)])

    @pl.when(c == 0)
    def _():
      copy_out(aggs_hbm)

    @pl.when(c == 1)
    def _():
      copy_out(aggf_hbm)

  return k(hs, hf, gidx, sidx, zeros)


_BLK = 1000


def _tc_dense_body(aggs_ref, hs_ref, aggf_ref, hf_ref,
                   wns, wss, bs1, wnf, wsf, bf1, os_ref, of_ref):
  os_ref[...] = jnp.maximum(
      jnp.dot(aggs_ref[...], wns[...], preferred_element_type=jnp.float32)
      + jnp.dot(hs_ref[...], wss[...], preferred_element_type=jnp.float32)
      + bs1[...], 0.0)
  of_ref[...] = jnp.maximum(
      jnp.dot(aggf_ref[...], wnf[...], preferred_element_type=jnp.float32)
      + jnp.dot(hf_ref[...], wsf[...], preferred_element_type=jnp.float32)
      + bf1[...], 0.0)


def _tc_dense(aggs, hs, aggf, hf, wns, wss, bs1, wnf, wsf, bf1):
  row_spec = pl.BlockSpec((_BLK, D), lambda i: (i, 0))
  w_spec = pl.BlockSpec((D, D), lambda i: (0, 0))
  b_spec = pl.BlockSpec((1, D), lambda i: (0, 0))
  return pl.pallas_call(
      _tc_dense_body,
      grid=(N // _BLK,),
      in_specs=[row_spec, row_spec, row_spec, row_spec,
                w_spec, w_spec, b_spec, w_spec, w_spec, b_spec],
      out_specs=[row_spec, row_spec],
      out_shape=[jax.ShapeDtypeStruct((N, D), jnp.float32)] * 2,
  )(aggs, hs, aggf, hf, wns, wss, bs1, wnf, wsf, bf1)


_PPT = P // NS


def _sc_po_gather(hs, hf, pos):
  mesh = plsc.VectorSubcoreMesh(core_axis_name="c", subcore_axis_name="s")

  @functools.partial(
      pl.kernel,
      out_type=[jax.ShapeDtypeStruct((P, D), jnp.float32)] * 2,
      mesh=mesh,
      scratch_types=[
          pltpu.VMEM((_PPT,), jnp.int32),
          pltpu.VMEM((_PPT, D), jnp.float32),
          pltpu.SemaphoreType.DMA,
      ],
  )
  def k(hs_hbm, hf_hbm, pos_hbm, embs_hbm, embf_hbm, pidx, rows, sem):
    c = lax.axis_index("c")
    s = lax.axis_index("s")
    base = s * _PPT
    pltpu.sync_copy(pos_hbm.at[pl.ds(base, _PPT)], pidx)

    @pl.when(c == 0)
    def _():
      pltpu.async_copy(hs_hbm.at[pidx], rows, sem).wait()
      pltpu.sync_copy(rows, embs_hbm.at[pl.ds(base, _PPT)])

    @pl.when(c == 1)
    def _():
      pltpu.async_copy(hf_hbm.at[pidx], rows, sem).wait()
      pltpu.sync_copy(rows, embf_hbm.at[pl.ds(base, _PPT)])

  return k(hs, hf, pos)


def _tc_mlp_body(es_ref, ef_ref, w1s, w1f, b1r, w2r, b2r, w3r, b3r, out_ref):
  h = jnp.maximum(
      jnp.dot(es_ref[...], w1s[...], preferred_element_type=jnp.float32)
      + jnp.dot(ef_ref[...], w1f[...], preferred_element_type=jnp.float32)
      + b1r[...], 0.0)
  h = jnp.maximum(
      jnp.dot(h, w2r[...], preferred_element_type=jnp.float32) + b2r[...], 0.0)
  out_ref[...] = (
      jnp.dot(h, w3r[...], preferred_element_type=jnp.float32) + b3r[...])


def _tc_mlp(embs, embf, w1s, w1f, b1, w2, b2, w3p, b3p):
  return pl.pallas_call(
      _tc_mlp_body,
      out_shape=jax.ShapeDtypeStruct((P, 128), jnp.float32),
  )(embs, embf, w1s, w1f, b1, w2, b2, w3p, b3p)


def kernel(x, edge_index, POs, Wn_s, Wself_s, b_s, Wn_f, Wself_f, b_f,
           W1, b1, W2, b2, W3, b3):
  src = edge_index[0]
  dst = edge_index[1]
  pad = NS * EPT_PAD - E
  gpad = jnp.zeros((pad,), jnp.int32)
  spad = jnp.full((pad,), N, jnp.int32)
  gidx = jnp.stack([jnp.concatenate([src, gpad]),
                    jnp.concatenate([dst, gpad])]).reshape(NC, NS, NSUP, SUP, CH)
  sidx = jnp.stack([jnp.concatenate([dst, spad]),
                    jnp.concatenate([src, spad])]).reshape(NC, NS, NSUP, SUP, CH)
  zeros = jnp.zeros((RPT_LAST, D), jnp.float32)

  hs = x
  hf = x
  for l in range(LAYERS):
    aggs, aggf = _sc_agg(hs, hf, gidx, sidx, zeros)
    hs, hf = _tc_dense(aggs, hs, aggf, hf,
                       Wn_s[l], Wself_s[l], b_s[l].reshape(1, D),
                       Wn_f[l], Wself_f[l], b_f[l].reshape(1, D))

  embs, embf = _sc_po_gather(hs, hf, POs)
  w3p = jnp.zeros((MLP_DIM, 128), jnp.float32).at[:, :NACT].set(W3)
  b3p = jnp.zeros((1, 128), jnp.float32).at[:, :NACT].set(b3.reshape(1, NACT))
  y = _tc_mlp(embs, embf, W1[:D], W1[D:], b1.reshape(1, MLP_DIM),
              W2, b2.reshape(1, MLP_DIM), w3p, b3p)
  return y[:, :NACT]
